# baseline (speedup 1.0000x reference)
import functools

import jax
import jax.numpy as jnp
from jax import lax
from jax.experimental import pallas as pl
from jax.experimental.pallas import tpu as pltpu
from jax.experimental.pallas import tpu_sc as plsc

F32 = jnp.float32


def _pack_pairs(lo, hi):
    lo16 = lax.bitcast_convert_type(lo.astype(jnp.bfloat16), jnp.uint16)
    hi16 = lax.bitcast_convert_type(hi.astype(jnp.bfloat16), jnp.uint16)
    word = lo16.astype(jnp.uint32) | (hi16.astype(jnp.uint32) << 16)
    return lax.bitcast_convert_type(word, jnp.int32)


def _prep1_body(node_ref, ncw_ref, ed_ref, w1_ref, b1_ref, w2_ref, b2_ref,
                sptT_ref, p_ref, hm_ref, sptp_ref):
    p_ref[...] = jnp.dot(node_ref[...], ncw_ref[...],
                         preferred_element_type=F32)
    a = jnp.maximum(
        jnp.dot(ed_ref[...], w1_ref[...], preferred_element_type=F32)
        + b1_ref[...], 0.0)
    hm_ref[...] = jnp.dot(a, w2_ref[...], preferred_element_type=F32) + b2_ref[...]
    s3 = sptT_ref[...].T.reshape(16, 2, 512)
    sptp_ref[...] = _pack_pairs(s3[:, 0, :], s3[:, 1, :])


def _prep1(node_data, nc_W, ed_pad, w1_pad, b1, w2, b2, sptT, BN, BL, H,
           interpret=False):
    return pl.pallas_call(
        _prep1_body,
        out_shape=(jax.ShapeDtypeStruct((BN, H), F32),
                   jax.ShapeDtypeStruct((BL, H), F32),
                   jax.ShapeDtypeStruct((H // 2, 512), jnp.int32)),
        interpret=interpret,
    )(node_data, nc_W, ed_pad, w1_pad, b1, w2, b2, sptT)


def _prep2_body(p_ref, hm_ref, g_ref, ncb_ref, wt_ref, t_ref):
    c = pl.program_id(0)
    src = g_ref[0, pl.ds(c * 1024, 1024)]
    dst = g_ref[1, pl.ds(c * 1024, 1024)]
    viota = lax.broadcasted_iota(jnp.int32, (1024, 2048), 1)
    oh = ((src[:, None] == viota).astype(F32)
          + (dst[:, None] == viota).astype(F32))
    nodes = jnp.dot(oh, p_ref[...], preferred_element_type=F32)
    ecat = nodes + hm_ref[...] + ncb_ref[...]
    ttT = jnp.dot(wt_ref[...], ecat.T, preferred_element_type=F32)
    t3 = ttT.reshape(80, 2, 1024)
    packed = _pack_pairs(t3[:, 0, :], t3[:, 1, :])
    t_ref[0, :, :512] = packed[:, :512]
    t_ref[1, :, :512] = packed[:, 512:]
    t_ref[:, :, 512:] = jnp.zeros_like(t_ref[:, :, 512:])


def _prep2(P, hm, graph3, ncb, wT, B, L, H, MD, interpret=False):
    nchunks = (B * L) // 1024
    return pl.pallas_call(
        _prep2_body,
        grid=(nchunks,),
        in_specs=[
            pl.BlockSpec((P.shape[0], H), lambda c: (0, 0)),
            pl.BlockSpec((1024, H), lambda c: (c, 0)),
            pl.BlockSpec((2, B * L), lambda c: (0, 0)),
            pl.BlockSpec((1, H), lambda c: (0, 0)),
            pl.BlockSpec((MD * H, H), lambda c: (0, 0)),
        ],
        out_specs=pl.BlockSpec((2, MD * H // 2, L + 1), lambda c: (c, 0, 0)),
        out_shape=jax.ShapeDtypeStruct((B, MD * H // 2, L + 1), jnp.int32),
        interpret=interpret,
    )(P, hm, graph3, ncb, wT)


def _sc_gather_body(t_hbm, spt_hbm, sp_hbm, ep_hbm, rtab_hbm, out_hbm,
                    t_v, spt_v, rtab_v, sp_v, ep_v, out_v):
    cid = lax.axis_index("c")
    sid = lax.axis_index("s")
    wid = sid * 2 + cid
    b = wid // 2
    iq = wid % 2

    pltpu.sync_copy(t_hbm.at[pl.ds(b * 41040, 41040)], t_v)
    pltpu.sync_copy(spt_hbm, spt_v)
    pltpu.sync_copy(rtab_hbm, rtab_v)

    row0 = iq * 64
    iota5 = lax.broadcasted_iota(jnp.int32, (16,), 0) * 5

    @pl.loop(0, 16)
    def _chunk(ci):
        i0 = row0 + ci * 4
        pltpu.sync_copy(sp_hbm.at[pl.ds(b * 16384 + i0 * 128, 512)], sp_v)
        pltpu.sync_copy(ep_hbm.at[pl.ds(b * 81920 + i0 * 640, 2560)], ep_v)

        for r in range(4):
            @pl.loop(0, 8)
            def _grp(jv, r=r):
                spi = sp_v[pl.ds(r * 128 + jv * 16, 16)]
                spc = jnp.clip(spi - 1, 1, 5)
                recip = plsc.load_gather(rtab_v, [spc])
                ep_base = iota5 + (r * 640 + jv * 80)
                ebs = [plsc.load_gather(ep_v, [ep_base + d]) for d in range(5)]
                recip2 = plsc.pack(recip, recip,
                                   format=plsc.PackFormat.INTERLEAVED)
                for hp in range(16):
                    wds = [plsc.load_gather(t_v, [ebs[d] + (d * 16 + hp) * 513])
                           for d in range(5)]
                    sw = plsc.load_gather(spt_v, [spi + hp * 512])
                    bv = [plsc.bitcast(w, jnp.bfloat16) for w in wds]
                    acc = ((bv[0] + bv[1]) + (bv[2] + bv[3])) + bv[4]
                    val = plsc.bitcast(sw, jnp.bfloat16) + recip2 * acc
                    vlo, vhi = plsc.unpack(val,
                                           format=plsc.PackFormat.INTERLEAVED)
                    out_v[2 * hp, r, pl.ds(jv * 16, 16)] = vlo
                    out_v[2 * hp + 1, r, pl.ds(jv * 16, 16)] = vhi

        pltpu.sync_copy(out_v, out_hbm.at[b, :, pl.ds(i0, 4), :])


def _sc_gather(t2, sptf, spf, epf, rtab, B, interpret=False):
    mesh = plsc.VectorSubcoreMesh(core_axis_name="c", subcore_axis_name="s",
                                  num_cores=2, num_subcores=16)
    f = pl.kernel(
        _sc_gather_body,
        out_type=jax.ShapeDtypeStruct((B, 32, 128, 128), F32),
        mesh=mesh,
        scratch_types=[
            pltpu.VMEM((41040,), jnp.int32),
            pltpu.VMEM((8192,), jnp.int32),
            pltpu.VMEM((16,), F32),
            pltpu.VMEM((512,), jnp.int32),
            pltpu.VMEM((2560,), jnp.int32),
            pltpu.VMEM((32, 4, 128), F32),
        ],
        compiler_params=pltpu.CompilerParams(needs_layout_passes=False),
        interpret=interpret,
    )
    return f(t2, sptf, spf, epf, rtab)


def _asm_body(ab_ref, d2_ref, a3_ref, g_ref, w2_ref, w3_ref, bias_ref,
              t_ref, out_ref):
    S = (jnp.dot(d2_ref[0], w2_ref[...], preferred_element_type=F32)
         + jnp.dot(a3_ref[0], w3_ref[...], preferred_element_type=F32)
         + bias_ref[...])
    ST = jnp.transpose(S.reshape(128, 128, 32), (2, 0, 1))
    inner = ST + g_ref[0]
    t = t_ref[...]
    t_col3 = jnp.broadcast_to(t.reshape(32, 1, 1), (32, 128, 1))
    t_row3 = jnp.broadcast_to(t.reshape(32, 1, 1), (32, 1, 129))
    m = jnp.concatenate([t_col3, inner], axis=2)
    m = jnp.concatenate([t_row3, m], axis=1)
    out_ref[0] = m + 2.0 * ab_ref[...]


def _assemble(ab, d2r, a3r, gath, d2_W, a3_W, bias2, t_col, B, H, N,
              interpret=False):
    return pl.pallas_call(
        _asm_body,
        grid=(B,),
        in_specs=[
            pl.BlockSpec((1, N + 1, N + 1), lambda b: (b, 0, 0)),
            pl.BlockSpec((1, N * N, H), lambda b: (b, 0, 0)),
            pl.BlockSpec((1, N * N, H), lambda b: (b, 0, 0)),
            pl.BlockSpec((1, H, N, N), lambda b: (b, 0, 0, 0)),
            pl.BlockSpec((H, H), lambda b: (0, 0)),
            pl.BlockSpec((H, H), lambda b: (0, 0)),
            pl.BlockSpec((1, H), lambda b: (0, 0)),
            pl.BlockSpec((H, 1), lambda b: (0, 0)),
        ],
        out_specs=pl.BlockSpec((1, H, N + 1, N + 1), lambda b: (b, 0, 0, 0)),
        out_shape=jax.ShapeDtypeStruct((B, H, N + 1, N + 1), F32),
        interpret=interpret,
    )(ab, d2r, a3r, gath, d2_W, a3_W, bias2, t_col)


def kernel(attn_bias, spatial_pos, d2_dist, a3_dist, edge_data, edge_path,
           edge_padding_mask, graph, node_data, spatial_pos_table, gt_vd,
           d2_W, d2_b, a3_W, a3_b, curv_W1, curv_b1, curv_W2, curv_b2,
           nc_W, nc_b, edge_dis_weight, interpret=False):
    B, N = edge_path.shape[0], edge_path.shape[1]
    MD = edge_path.shape[-1]
    H = spatial_pos_table.shape[1]
    L = edge_padding_mask.shape[1]

    ed_pad = jnp.pad(edge_data.reshape(B * L, 7), ((0, 0), (0, 1)))
    w1_pad = jnp.pad(curv_W1, ((0, 1), (0, 0)))
    b1 = curv_b1.reshape(1, -1)
    b2 = curv_b2.reshape(1, -1)
    ncb = nc_b.reshape(1, H)
    w = edge_dis_weight.reshape(-1, H, H)[:MD]
    wT = w.transpose(0, 2, 1).reshape(MD * H, H)

    P, hm, sptp = _prep1(node_data, nc_W, ed_pad, w1_pad, b1, curv_W2, b2,
                         spatial_pos_table, B * N, B * L, H,
                         interpret=interpret)
    T = _prep2(P, hm, graph, ncb, wT, B, L, H, MD, interpret=interpret)

    t2 = T.reshape(B * (MD * H // 2) * (L + 1))
    sptf = sptp.reshape(-1)
    spf = spatial_pos.reshape(-1)
    epf = edge_path.reshape(-1)
    rtab = (1.0 / jnp.clip(jnp.arange(16, dtype=F32), 1.0, 5.0)).astype(F32)

    bias2 = (d2_b + a3_b).reshape(1, H)
    t_col = gt_vd.reshape(1, H).T
    d2r = d2_dist.reshape(B, N * N, H)
    a3r = a3_dist.reshape(B, N * N, H)

    gath = _sc_gather(t2, sptf, spf, epf, rtab, B, interpret=interpret)
    return _assemble(attn_bias, d2r, a3r, gath, d2_W, a3_W, bias2,
                     t_col, B, H, N, interpret=interpret)


# speedup vs baseline: 1.1198x; 1.1198x over previous
import functools

import jax
import jax.numpy as jnp
from jax import lax
from jax.experimental import pallas as pl
from jax.experimental.pallas import tpu as pltpu
from jax.experimental.pallas import tpu_sc as plsc

F32 = jnp.float32


def _pack_pairs(lo, hi):
    lo16 = lax.bitcast_convert_type(lo.astype(jnp.bfloat16), jnp.uint16)
    hi16 = lax.bitcast_convert_type(hi.astype(jnp.bfloat16), jnp.uint16)
    word = lo16.astype(jnp.uint32) | (hi16.astype(jnp.uint32) << 16)
    return lax.bitcast_convert_type(word, jnp.int32)


def _prep1_body(node_ref, ncw_ref, ed_ref, w1_ref, b1_ref, w2_ref, b2_ref,
                sptT_ref, p_ref, hm_ref, sptp_ref):
    p_ref[...] = jnp.dot(node_ref[...], ncw_ref[...],
                         preferred_element_type=F32)
    a = jnp.maximum(
        jnp.dot(ed_ref[...], w1_ref[...], preferred_element_type=F32)
        + b1_ref[...], 0.0)
    hm_ref[...] = jnp.dot(a, w2_ref[...], preferred_element_type=F32) + b2_ref[...]
    s3 = sptT_ref[...].T.reshape(16, 2, 512)
    sptp_ref[...] = _pack_pairs(s3[:, 0, :], s3[:, 1, :])


def _prep1(node_data, nc_W, ed_pad, w1_pad, b1, w2, b2, sptT, BN, BL, H,
           interpret=False):
    return pl.pallas_call(
        _prep1_body,
        out_shape=(jax.ShapeDtypeStruct((BN, H), F32),
                   jax.ShapeDtypeStruct((BL, H), F32),
                   jax.ShapeDtypeStruct((H // 2, 512), jnp.int32)),
        interpret=interpret,
    )(node_data, nc_W, ed_pad, w1_pad, b1, w2, b2, sptT)


def _prep2_body(p_ref, hm_ref, g_ref, ncb_ref, wt_ref, t_ref):
    c = pl.program_id(0)
    src = g_ref[0, pl.ds(c * 1024, 1024)]
    dst = g_ref[1, pl.ds(c * 1024, 1024)]
    viota = lax.broadcasted_iota(jnp.int32, (1024, 2048), 1)
    oh = ((src[:, None] == viota).astype(F32)
          + (dst[:, None] == viota).astype(F32))
    nodes = jnp.dot(oh, p_ref[...], preferred_element_type=F32)
    ecat = nodes + hm_ref[...] + ncb_ref[...]
    ttT = jnp.dot(wt_ref[...], ecat.T, preferred_element_type=F32)
    t3 = ttT.reshape(80, 2, 1024)
    packed = _pack_pairs(t3[:, 0, :], t3[:, 1, :])
    t_ref[0, :, :512] = packed[:, :512]
    t_ref[1, :, :512] = packed[:, 512:]
    t_ref[:, :, 512:] = jnp.zeros_like(t_ref[:, :, 512:])


def _prep2(P, hm, graph3, ncb, wT, B, L, H, MD, interpret=False):
    nchunks = (B * L) // 1024
    return pl.pallas_call(
        _prep2_body,
        grid=(nchunks,),
        in_specs=[
            pl.BlockSpec((P.shape[0], H), lambda c: (0, 0)),
            pl.BlockSpec((1024, H), lambda c: (c, 0)),
            pl.BlockSpec((2, B * L), lambda c: (0, 0)),
            pl.BlockSpec((1, H), lambda c: (0, 0)),
            pl.BlockSpec((MD * H, H), lambda c: (0, 0)),
        ],
        out_specs=pl.BlockSpec((2, MD * H // 2, L + 1), lambda c: (c, 0, 0)),
        out_shape=jax.ShapeDtypeStruct((B, MD * H // 2, L + 1), jnp.int32),
        interpret=interpret,
    )(P, hm, graph3, ncb, wT)


def _sc_gather_body(t_hbm, spt_hbm, sp_hbm, ep_hbm, rtab_hbm, out_hbm,
                    t_v, spt_v, rtab_v, sp_v, ep_v, out_v):
    cid = lax.axis_index("c")
    sid = lax.axis_index("s")
    wid = sid * 2 + cid
    b = wid // 2
    iq = wid % 2

    pltpu.sync_copy(t_hbm.at[b], t_v)
    pltpu.sync_copy(spt_hbm, spt_v)
    pltpu.sync_copy(rtab_hbm, rtab_v)

    row0 = iq * 64
    iota5 = lax.broadcasted_iota(jnp.int32, (16,), 0) * 5

    @pl.loop(0, 16)
    def _chunk(ci):
        i0 = row0 + ci * 4
        pltpu.sync_copy(sp_hbm.at[b, pl.ds(i0 * 128, 512)], sp_v)
        pltpu.sync_copy(ep_hbm.at[b, pl.ds(i0 * 640, 2560)], ep_v)

        for r in range(4):
            @pl.loop(0, 8)
            def _grp(jv, r=r):
                spi = sp_v[pl.ds(r * 128 + jv * 16, 16)]
                spc = jnp.clip(spi - 1, 1, 5)
                recip = plsc.load_gather(rtab_v, [spc])
                ep_base = iota5 + (r * 640 + jv * 80)
                ebs = [plsc.load_gather(ep_v, [ep_base + d]) for d in range(5)]
                recip2 = plsc.pack(recip, recip,
                                   format=plsc.PackFormat.INTERLEAVED)
                for hp in range(16):
                    wds = [plsc.load_gather(t_v, [ebs[d] + (d * 16 + hp) * 513])
                           for d in range(5)]
                    sw = plsc.load_gather(spt_v, [spi + hp * 512])
                    bv = [plsc.bitcast(w, jnp.bfloat16) for w in wds]
                    acc = ((bv[0] + bv[1]) + (bv[2] + bv[3])) + bv[4]
                    val = plsc.bitcast(sw, jnp.bfloat16) + recip2 * acc
                    vlo, vhi = plsc.unpack(val,
                                           format=plsc.PackFormat.INTERLEAVED)
                    out_v[2 * hp, r, pl.ds(jv * 16, 16)] = vlo
                    out_v[2 * hp + 1, r, pl.ds(jv * 16, 16)] = vhi

        pltpu.sync_copy(out_v, out_hbm.at[b, :, pl.ds(i0, 4), :])


def _sc_gather(t2, sptf, spf, epf, rtab, B, interpret=False):
    mesh = plsc.VectorSubcoreMesh(core_axis_name="c", subcore_axis_name="s",
                                  num_cores=2, num_subcores=16)
    f = pl.kernel(
        _sc_gather_body,
        out_type=jax.ShapeDtypeStruct((B, 32, 128, 128), F32),
        mesh=mesh,
        scratch_types=[
            pltpu.VMEM((41040,), jnp.int32),
            pltpu.VMEM((8192,), jnp.int32),
            pltpu.VMEM((16,), F32),
            pltpu.VMEM((512,), jnp.int32),
            pltpu.VMEM((2560,), jnp.int32),
            pltpu.VMEM((32, 4, 128), F32),
        ],
        compiler_params=pltpu.CompilerParams(needs_layout_passes=False),
        interpret=interpret,
    )
    return f(t2, sptf, spf, epf, rtab)


def _asm_body(ab_ref, d2_ref, a3_ref, g_ref, w2_ref, w3_ref, bias_ref,
              t_ref, out_ref):
    S = (jnp.dot(d2_ref[0], w2_ref[...], preferred_element_type=F32)
         + jnp.dot(a3_ref[0], w3_ref[...], preferred_element_type=F32)
         + bias_ref[...])
    ST = jnp.transpose(S.reshape(128, 128, 32), (2, 0, 1))
    inner = ST + g_ref[0]
    t = t_ref[...]
    t_col3 = jnp.broadcast_to(t.reshape(32, 1, 1), (32, 128, 1))
    t_row3 = jnp.broadcast_to(t.reshape(32, 1, 1), (32, 1, 129))
    m = jnp.concatenate([t_col3, inner], axis=2)
    m = jnp.concatenate([t_row3, m], axis=1)
    out_ref[0] = m + 2.0 * ab_ref[...]


def _assemble(ab, d2r, a3r, gath, d2_W, a3_W, bias2, t_col, B, H, N,
              interpret=False):
    return pl.pallas_call(
        _asm_body,
        grid=(B,),
        in_specs=[
            pl.BlockSpec((1, N + 1, N + 1), lambda b: (b, 0, 0)),
            pl.BlockSpec((1, N * N, H), lambda b: (b, 0, 0)),
            pl.BlockSpec((1, N * N, H), lambda b: (b, 0, 0)),
            pl.BlockSpec((1, H, N, N), lambda b: (b, 0, 0, 0)),
            pl.BlockSpec((H, H), lambda b: (0, 0)),
            pl.BlockSpec((H, H), lambda b: (0, 0)),
            pl.BlockSpec((1, H), lambda b: (0, 0)),
            pl.BlockSpec((H, 1), lambda b: (0, 0)),
        ],
        out_specs=pl.BlockSpec((1, H, N + 1, N + 1), lambda b: (b, 0, 0, 0)),
        out_shape=jax.ShapeDtypeStruct((B, H, N + 1, N + 1), F32),
        interpret=interpret,
    )(ab, d2r, a3r, gath, d2_W, a3_W, bias2, t_col)


def kernel(attn_bias, spatial_pos, d2_dist, a3_dist, edge_data, edge_path,
           edge_padding_mask, graph, node_data, spatial_pos_table, gt_vd,
           d2_W, d2_b, a3_W, a3_b, curv_W1, curv_b1, curv_W2, curv_b2,
           nc_W, nc_b, edge_dis_weight, interpret=False):
    B, N = edge_path.shape[0], edge_path.shape[1]
    MD = edge_path.shape[-1]
    H = spatial_pos_table.shape[1]
    L = edge_padding_mask.shape[1]

    ed_pad = jnp.pad(edge_data.reshape(B * L, 7), ((0, 0), (0, 1)))
    w1_pad = jnp.pad(curv_W1, ((0, 1), (0, 0)))
    b1 = curv_b1.reshape(1, -1)
    b2 = curv_b2.reshape(1, -1)
    ncb = nc_b.reshape(1, H)
    w = edge_dis_weight.reshape(-1, H, H)[:MD]
    wT = w.transpose(0, 2, 1).reshape(MD * H, H)

    P, hm, sptp = _prep1(node_data, nc_W, ed_pad, w1_pad, b1, curv_W2, b2,
                         spatial_pos_table, B * N, B * L, H,
                         interpret=interpret)
    T = _prep2(P, hm, graph, ncb, wT, B, L, H, MD, interpret=interpret)

    t2 = T.reshape(B, (MD * H // 2) * (L + 1))
    sptf = sptp.reshape(-1)
    spf = spatial_pos.reshape(B, N * N)
    epf = edge_path.reshape(B, N * N * MD)
    rtab = (1.0 / jnp.clip(jnp.arange(16, dtype=F32), 1.0, 5.0)).astype(F32)

    bias2 = (d2_b + a3_b).reshape(1, H)
    t_col = gt_vd.reshape(1, H).T
    d2r = d2_dist.reshape(B, N * N, H)
    a3r = a3_dist.reshape(B, N * N, H)

    gath = _sc_gather(t2, sptf, spf, epf, rtab, B, interpret=interpret)
    return _assemble(attn_bias, d2r, a3r, gath, d2_W, a3_W, bias2,
                     t_col, B, H, N, interpret=interpret)


# speedup vs baseline: 1.1443x; 1.0219x over previous
import functools

import jax
import jax.numpy as jnp
from jax import lax
from jax.experimental import pallas as pl
from jax.experimental.pallas import tpu as pltpu
from jax.experimental.pallas import tpu_sc as plsc

F32 = jnp.float32


def _pack_pairs(lo, hi):
    lo16 = lax.bitcast_convert_type(lo.astype(jnp.bfloat16), jnp.uint16)
    hi16 = lax.bitcast_convert_type(hi.astype(jnp.bfloat16), jnp.uint16)
    word = lo16.astype(jnp.uint32) | (hi16.astype(jnp.uint32) << 16)
    return lax.bitcast_convert_type(word, jnp.int32)


def _prep1_body(node_ref, ncw_ref, ed_ref, w1_ref, b1_ref, w2_ref, b2_ref,
                sptT_ref, p_ref, hm_ref, sptp_ref):
    p_ref[...] = jnp.dot(node_ref[...], ncw_ref[...],
                         preferred_element_type=F32)
    a = jnp.maximum(
        jnp.dot(ed_ref[...], w1_ref[...], preferred_element_type=F32)
        + b1_ref[...], 0.0)
    hm_ref[...] = jnp.dot(a, w2_ref[...], preferred_element_type=F32) + b2_ref[...]
    s3 = sptT_ref[...].T.reshape(16, 2, 512)
    sptp_ref[...] = _pack_pairs(s3[:, 0, :], s3[:, 1, :])


def _prep1(node_data, nc_W, ed_pad, w1_pad, b1, w2, b2, sptT, BN, BL, H,
           interpret=False):
    return pl.pallas_call(
        _prep1_body,
        out_shape=(jax.ShapeDtypeStruct((BN, H), F32),
                   jax.ShapeDtypeStruct((BL, H), F32),
                   jax.ShapeDtypeStruct((H // 2, 512), jnp.int32)),
        interpret=interpret,
    )(node_data, nc_W, ed_pad, w1_pad, b1, w2, b2, sptT)


def _prep2_body(p_ref, hm_ref, g_ref, ncb_ref, wt_ref, t_ref):
    c = pl.program_id(0)
    src = g_ref[0, pl.ds(c * 1024, 1024)]
    dst = g_ref[1, pl.ds(c * 1024, 1024)]
    viota = lax.broadcasted_iota(jnp.int32, (1024, 2048), 1)
    oh = ((src[:, None] == viota).astype(F32)
          + (dst[:, None] == viota).astype(F32))
    nodes = jnp.dot(oh, p_ref[...], preferred_element_type=F32)
    ecat = nodes + hm_ref[...] + ncb_ref[...]
    ttT = jnp.dot(wt_ref[...], ecat.T, preferred_element_type=F32)
    t3 = ttT.reshape(80, 2, 1024)
    packed = _pack_pairs(t3[:, 0, :], t3[:, 1, :])
    t_ref[0, :, :512] = packed[:, :512]
    t_ref[1, :, :512] = packed[:, 512:]
    t_ref[:, :, 512:] = jnp.zeros_like(t_ref[:, :, 512:])


def _prep2(P, hm, graph3, ncb, wT, B, L, H, MD, interpret=False):
    nchunks = (B * L) // 1024
    return pl.pallas_call(
        _prep2_body,
        grid=(nchunks,),
        in_specs=[
            pl.BlockSpec((P.shape[0], H), lambda c: (0, 0)),
            pl.BlockSpec((1024, H), lambda c: (c, 0)),
            pl.BlockSpec((2, B * L), lambda c: (0, 0)),
            pl.BlockSpec((1, H), lambda c: (0, 0)),
            pl.BlockSpec((MD * H, H), lambda c: (0, 0)),
        ],
        out_specs=pl.BlockSpec((2, MD * H // 2, L + 1), lambda c: (c, 0, 0)),
        out_shape=jax.ShapeDtypeStruct((B, MD * H // 2, L + 1), jnp.int32),
        interpret=interpret,
    )(P, hm, graph3, ncb, wT)


def _sc_gather_body(t_hbm, spt_hbm, sp_hbm, ep_hbm, rtab_hbm, out_hbm,
                    t_v, spt_v, rtab_v, sp_v, ep_v, out_v):
    cid = lax.axis_index("c")
    sid = lax.axis_index("s")
    wid = sid * 2 + cid
    b = wid // 2
    iq = wid % 2

    pltpu.sync_copy(t_hbm.at[b], t_v)
    pltpu.sync_copy(spt_hbm, spt_v)
    pltpu.sync_copy(rtab_hbm, rtab_v)

    row0 = iq * 64
    iota5 = lax.broadcasted_iota(jnp.int32, (16,), 0) * 5

    pltpu.sync_copy(ep_hbm.at[b, pl.ds(row0 * 640, 40960)], ep_v)

    @pl.loop(0, 8)
    def _chunk(ci):
        i0 = row0 + ci * 8
        pltpu.sync_copy(sp_hbm.at[b, pl.ds(i0 * 128, 1024)], sp_v)

        for r in range(8):
            @pl.loop(0, 8)
            def _grp(jv, r=r):
                spi = sp_v[pl.ds(r * 128 + jv * 16, 16)]
                spc = jnp.clip(spi - 1, 1, 5)
                recip = plsc.load_gather(rtab_v, [spc])
                ep_base = iota5 + ((ci * 8 + r) * 640 + jv * 80)
                ebs = [plsc.load_gather(ep_v, [ep_base + d]) for d in range(5)]
                recip2 = plsc.pack(recip, recip,
                                   format=plsc.PackFormat.INTERLEAVED)
                for hp in range(16):
                    wds = [plsc.load_gather(t_v, [ebs[d] + (d * 16 + hp) * 513])
                           for d in range(5)]
                    sw = plsc.load_gather(spt_v, [spi + hp * 512])
                    bv = [plsc.bitcast(w, jnp.bfloat16) for w in wds]
                    acc = ((bv[0] + bv[1]) + (bv[2] + bv[3])) + bv[4]
                    val = plsc.bitcast(sw, jnp.bfloat16) + recip2 * acc
                    vlo, vhi = plsc.unpack(val,
                                           format=plsc.PackFormat.INTERLEAVED)
                    out_v[2 * hp, r, pl.ds(jv * 16, 16)] = vlo
                    out_v[2 * hp + 1, r, pl.ds(jv * 16, 16)] = vhi

        pltpu.sync_copy(out_v, out_hbm.at[b, :, pl.ds(i0, 8), :])


def _sc_gather(t2, sptf, spf, epf, rtab, B, interpret=False):
    mesh = plsc.VectorSubcoreMesh(core_axis_name="c", subcore_axis_name="s",
                                  num_cores=2, num_subcores=16)
    f = pl.kernel(
        _sc_gather_body,
        out_type=jax.ShapeDtypeStruct((B, 32, 128, 128), F32),
        mesh=mesh,
        scratch_types=[
            pltpu.VMEM((41040,), jnp.int32),
            pltpu.VMEM((8192,), jnp.int32),
            pltpu.VMEM((16,), F32),
            pltpu.VMEM((1024,), jnp.int32),
            pltpu.VMEM((40960,), jnp.int32),
            pltpu.VMEM((32, 8, 128), F32),
        ],
        compiler_params=pltpu.CompilerParams(needs_layout_passes=False),
        interpret=interpret,
    )
    return f(t2, sptf, spf, epf, rtab)


def _asm_body(ab_ref, d2_ref, a3_ref, g_ref, w2_ref, w3_ref, bias_ref,
              t_ref, out_ref):
    S = (jnp.dot(d2_ref[0], w2_ref[...], preferred_element_type=F32)
         + jnp.dot(a3_ref[0], w3_ref[...], preferred_element_type=F32)
         + bias_ref[...])
    ST = jnp.transpose(S.reshape(128, 128, 32), (2, 0, 1))
    inner = ST + g_ref[0]
    t = t_ref[...]
    t_col3 = jnp.broadcast_to(t.reshape(32, 1, 1), (32, 128, 1))
    t_row3 = jnp.broadcast_to(t.reshape(32, 1, 1), (32, 1, 129))
    m = jnp.concatenate([t_col3, inner], axis=2)
    m = jnp.concatenate([t_row3, m], axis=1)
    out_ref[0] = m + 2.0 * ab_ref[...]


def _assemble(ab, d2r, a3r, gath, d2_W, a3_W, bias2, t_col, B, H, N,
              interpret=False):
    return pl.pallas_call(
        _asm_body,
        grid=(B,),
        in_specs=[
            pl.BlockSpec((1, N + 1, N + 1), lambda b: (b, 0, 0)),
            pl.BlockSpec((1, N * N, H), lambda b: (b, 0, 0)),
            pl.BlockSpec((1, N * N, H), lambda b: (b, 0, 0)),
            pl.BlockSpec((1, H, N, N), lambda b: (b, 0, 0, 0)),
            pl.BlockSpec((H, H), lambda b: (0, 0)),
            pl.BlockSpec((H, H), lambda b: (0, 0)),
            pl.BlockSpec((1, H), lambda b: (0, 0)),
            pl.BlockSpec((H, 1), lambda b: (0, 0)),
        ],
        out_specs=pl.BlockSpec((1, H, N + 1, N + 1), lambda b: (b, 0, 0, 0)),
        out_shape=jax.ShapeDtypeStruct((B, H, N + 1, N + 1), F32),
        interpret=interpret,
    )(ab, d2r, a3r, gath, d2_W, a3_W, bias2, t_col)


def kernel(attn_bias, spatial_pos, d2_dist, a3_dist, edge_data, edge_path,
           edge_padding_mask, graph, node_data, spatial_pos_table, gt_vd,
           d2_W, d2_b, a3_W, a3_b, curv_W1, curv_b1, curv_W2, curv_b2,
           nc_W, nc_b, edge_dis_weight, interpret=False):
    B, N = edge_path.shape[0], edge_path.shape[1]
    MD = edge_path.shape[-1]
    H = spatial_pos_table.shape[1]
    L = edge_padding_mask.shape[1]

    ed_pad = jnp.pad(edge_data.reshape(B * L, 7), ((0, 0), (0, 1)))
    w1_pad = jnp.pad(curv_W1, ((0, 1), (0, 0)))
    b1 = curv_b1.reshape(1, -1)
    b2 = curv_b2.reshape(1, -1)
    ncb = nc_b.reshape(1, H)
    w = edge_dis_weight.reshape(-1, H, H)[:MD]
    wT = w.transpose(0, 2, 1).reshape(MD * H, H)

    P, hm, sptp = _prep1(node_data, nc_W, ed_pad, w1_pad, b1, curv_W2, b2,
                         spatial_pos_table, B * N, B * L, H,
                         interpret=interpret)
    T = _prep2(P, hm, graph, ncb, wT, B, L, H, MD, interpret=interpret)

    t2 = T.reshape(B, (MD * H // 2) * (L + 1))
    sptf = sptp.reshape(-1)
    spf = spatial_pos.reshape(B, N * N)
    epf = edge_path.reshape(B, N * N * MD)
    rtab = (1.0 / jnp.clip(jnp.arange(16, dtype=F32), 1.0, 5.0)).astype(F32)

    bias2 = (d2_b + a3_b).reshape(1, H)
    t_col = gt_vd.reshape(1, H).T
    d2r = d2_dist.reshape(B, N * N, H)
    a3r = a3_dist.reshape(B, N * N, H)

    gath = _sc_gather(t2, sptf, spf, epf, rtab, B, interpret=interpret)
    return _assemble(attn_bias, d2r, a3r, gath, d2_W, a3_W, bias2,
                     t_col, B, H, N, interpret=interpret)


# speedup vs baseline: 1.1463x; 1.0017x over previous
import jax
import jax.numpy as jnp
from jax import lax
from jax.experimental import pallas as pl
from jax.experimental.pallas import tpu as pltpu
from jax.experimental.pallas import tpu_sc as plsc

F32 = jnp.float32


def _pack_pairs(lo, hi):
    lo16 = lax.bitcast_convert_type(lo.astype(jnp.bfloat16), jnp.uint16)
    hi16 = lax.bitcast_convert_type(hi.astype(jnp.bfloat16), jnp.uint16)
    word = lo16.astype(jnp.uint32) | (hi16.astype(jnp.uint32) << 16)
    return lax.bitcast_convert_type(word, jnp.int32)


def _prep1_body(node_ref, ncw_ref, ed_ref, w1_ref, b1_ref, w2_ref, b2_ref,
                sptT_ref, p_ref, hm_ref, sptp_ref):
    p_ref[...] = jnp.dot(node_ref[...], ncw_ref[...],
                         preferred_element_type=F32)
    a = jnp.maximum(
        jnp.dot(ed_ref[...], w1_ref[...], preferred_element_type=F32)
        + b1_ref[...], 0.0)
    hm_ref[...] = jnp.dot(a, w2_ref[...], preferred_element_type=F32) + b2_ref[...]
    s3 = sptT_ref[...].T.reshape(16, 2, 512)
    sptp_ref[...] = _pack_pairs(s3[:, 0, :], s3[:, 1, :])


def _prep1(node_data, nc_W, ed_pad, w1_pad, b1, w2, b2, sptT, BN, BL, H,
           interpret=False):
    return pl.pallas_call(
        _prep1_body,
        out_shape=(jax.ShapeDtypeStruct((BN, H), F32),
                   jax.ShapeDtypeStruct((BL, H), F32),
                   jax.ShapeDtypeStruct((H // 2, 512), jnp.int32)),
        interpret=interpret,
    )(node_data, nc_W, ed_pad, w1_pad, b1, w2, b2, sptT)


def _prep2_body(p_ref, hm_ref, g_ref, ncb_ref, wt_ref, t_ref):
    c = pl.program_id(0)
    src = g_ref[0, pl.ds(c * 1024, 1024)]
    dst = g_ref[1, pl.ds(c * 1024, 1024)]
    viota = lax.broadcasted_iota(jnp.int32, (1024, 2048), 1)
    oh = ((src[:, None] == viota).astype(F32)
          + (dst[:, None] == viota).astype(F32))
    nodes = jnp.dot(oh, p_ref[...], preferred_element_type=F32)
    ecat = nodes + hm_ref[...] + ncb_ref[...]
    ttT = jnp.dot(wt_ref[...], ecat.T, preferred_element_type=F32)
    t3 = ttT.reshape(80, 2, 1024)
    packed = _pack_pairs(t3[:, 0, :], t3[:, 1, :])
    t_ref[0, :, :512] = packed[:, :512]
    t_ref[1, :, :512] = packed[:, 512:]
    t_ref[:, :, 512:] = jnp.zeros_like(t_ref[:, :, 512:])


def _prep2(P, hm, graph3, ncb, wT, B, L, H, MD, interpret=False):
    nchunks = (B * L) // 1024
    return pl.pallas_call(
        _prep2_body,
        grid=(nchunks,),
        in_specs=[
            pl.BlockSpec((P.shape[0], H), lambda c: (0, 0)),
            pl.BlockSpec((1024, H), lambda c: (c, 0)),
            pl.BlockSpec((2, B * L), lambda c: (0, 0)),
            pl.BlockSpec((1, H), lambda c: (0, 0)),
            pl.BlockSpec((MD * H, H), lambda c: (0, 0)),
        ],
        out_specs=pl.BlockSpec((2, MD * H // 2, L + 1), lambda c: (c, 0, 0)),
        out_shape=jax.ShapeDtypeStruct((B, MD * H // 2, L + 1), jnp.int32),
        interpret=interpret,
    )(P, hm, graph3, ncb, wT)


def _sc_gather_body(t_hbm, spt_hbm, sp_hbm, ep_hbm, rtab_hbm, out_hbm,
                    t_v, spt_v, rtab_v, sp_v, ep_v, out_v):
    cid = lax.axis_index("c")
    sid = lax.axis_index("s")
    wid = sid * 2 + cid
    b = wid // 2
    iq = wid % 2

    pltpu.sync_copy(t_hbm.at[b], t_v)
    pltpu.sync_copy(spt_hbm, spt_v)
    pltpu.sync_copy(rtab_hbm, rtab_v)

    row0 = iq * 64
    iota5 = lax.broadcasted_iota(jnp.int32, (16,), 0) * 5

    pltpu.sync_copy(ep_hbm.at[b, pl.ds(row0 * 640, 40960)], ep_v)

    @pl.loop(0, 8)
    def _chunk(ci):
        i0 = row0 + ci * 8
        pltpu.sync_copy(sp_hbm.at[b, pl.ds(i0 * 128, 1024)], sp_v)

        for r in range(8):
            @pl.loop(0, 8)
            def _grp(jv, r=r):
                spi = sp_v[pl.ds(r * 128 + jv * 16, 16)]
                spc = jnp.clip(spi - 1, 1, 5)
                recip = plsc.load_gather(rtab_v, [spc])
                ep_base = iota5 + ((ci * 8 + r) * 640 + jv * 80)
                ebs = [plsc.load_gather(ep_v, [ep_base + d]) for d in range(5)]
                recip2 = plsc.pack(recip, recip,
                                   format=plsc.PackFormat.INTERLEAVED)
                for hp in range(16):
                    wds = [plsc.load_gather(t_v, [ebs[d] + (d * 16 + hp) * 513])
                           for d in range(5)]
                    sw = plsc.load_gather(spt_v, [spi + hp * 512])
                    bv = [plsc.bitcast(w, jnp.bfloat16) for w in wds]
                    acc = ((bv[0] + bv[1]) + (bv[2] + bv[3])) + bv[4]
                    val = plsc.bitcast(sw, jnp.bfloat16) + recip2 * acc
                    vlo, vhi = plsc.unpack(val,
                                           format=plsc.PackFormat.INTERLEAVED)
                    out_v[2 * hp, r, pl.ds(jv * 16, 16)] = vlo
                    out_v[2 * hp + 1, r, pl.ds(jv * 16, 16)] = vhi

        pltpu.sync_copy(out_v, out_hbm.at[b, :, pl.ds(i0, 8), :])


def _sc_gather(t2, sptf, spf, epf, rtab, B, interpret=False):
    mesh = plsc.VectorSubcoreMesh(core_axis_name="c", subcore_axis_name="s",
                                  num_cores=2, num_subcores=16)
    f = pl.kernel(
        _sc_gather_body,
        out_type=jax.ShapeDtypeStruct((B, 32, 128, 128), F32),
        mesh=mesh,
        scratch_types=[
            pltpu.VMEM((41040,), jnp.int32),
            pltpu.VMEM((8192,), jnp.int32),
            pltpu.VMEM((16,), F32),
            pltpu.VMEM((1024,), jnp.int32),
            pltpu.VMEM((40960,), jnp.int32),
            pltpu.VMEM((32, 8, 128), F32),
        ],
        compiler_params=pltpu.CompilerParams(needs_layout_passes=False),
        interpret=interpret,
    )
    return f(t2, sptf, spf, epf, rtab)


def _asm_body(ab_ref, d2_ref, a3_ref, g_ref, w2_ref, w3_ref, bias_ref,
              t_ref, out_ref):
    S = (jnp.dot(d2_ref[0], w2_ref[...], preferred_element_type=F32)
         + jnp.dot(a3_ref[0], w3_ref[...], preferred_element_type=F32)
         + bias_ref[...])
    ST = jnp.transpose(S.reshape(128, 128, 32), (2, 0, 1))
    inner = ST + g_ref[0]
    t = t_ref[...]
    t_col3 = jnp.broadcast_to(t.reshape(32, 1, 1), (32, 128, 1))
    t_row3 = jnp.broadcast_to(t.reshape(32, 1, 1), (32, 1, 129))
    m = jnp.concatenate([t_col3, inner], axis=2)
    m = jnp.concatenate([t_row3, m], axis=1)
    out_ref[0] = m + 2.0 * ab_ref[...]


def _assemble(ab, d2r, a3r, gath, d2_W, a3_W, bias2, t_col, B, H, N,
              interpret=False):
    return pl.pallas_call(
        _asm_body,
        grid=(B,),
        in_specs=[
            pl.BlockSpec((1, N + 1, N + 1), lambda b: (b, 0, 0)),
            pl.BlockSpec((1, N * N, H), lambda b: (b, 0, 0)),
            pl.BlockSpec((1, N * N, H), lambda b: (b, 0, 0)),
            pl.BlockSpec((1, H, N, N), lambda b: (b, 0, 0, 0)),
            pl.BlockSpec((H, H), lambda b: (0, 0)),
            pl.BlockSpec((H, H), lambda b: (0, 0)),
            pl.BlockSpec((1, H), lambda b: (0, 0)),
            pl.BlockSpec((H, 1), lambda b: (0, 0)),
        ],
        out_specs=pl.BlockSpec((1, H, N + 1, N + 1), lambda b: (b, 0, 0, 0)),
        out_shape=jax.ShapeDtypeStruct((B, H, N + 1, N + 1), F32),
        interpret=interpret,
    )(ab, d2r, a3r, gath, d2_W, a3_W, bias2, t_col)


def kernel(attn_bias, spatial_pos, d2_dist, a3_dist, edge_data, edge_path,
           edge_padding_mask, graph, node_data, spatial_pos_table, gt_vd,
           d2_W, d2_b, a3_W, a3_b, curv_W1, curv_b1, curv_W2, curv_b2,
           nc_W, nc_b, edge_dis_weight, interpret=False):
    B, N = edge_path.shape[0], edge_path.shape[1]
    MD = edge_path.shape[-1]
    H = spatial_pos_table.shape[1]
    L = edge_padding_mask.shape[1]

    ed_pad = jnp.pad(edge_data.reshape(B * L, 7), ((0, 0), (0, 1)))
    w1_pad = jnp.pad(curv_W1, ((0, 1), (0, 0)))
    b1 = curv_b1.reshape(1, -1)
    b2 = curv_b2.reshape(1, -1)
    ncb = nc_b.reshape(1, H)
    w = edge_dis_weight.reshape(-1, H, H)[:MD]
    wT = w.transpose(0, 2, 1).reshape(MD * H, H)

    P, hm, sptp = _prep1(node_data, nc_W, ed_pad, w1_pad, b1, curv_W2, b2,
                         spatial_pos_table, B * N, B * L, H,
                         interpret=interpret)
    T = _prep2(P, hm, graph, ncb, wT, B, L, H, MD, interpret=interpret)

    t2 = T.reshape(B, (MD * H // 2) * (L + 1))
    sptf = sptp.reshape(-1)
    spf = spatial_pos.reshape(B, N * N)
    epf = edge_path.reshape(B, N * N * MD)
    rtab = (1.0 / jnp.clip(jnp.arange(16, dtype=F32), 1.0, 5.0)).astype(F32)

    bias2 = (d2_b + a3_b).reshape(1, H)
    t_col = gt_vd.reshape(1, H).T
    d2r = d2_dist.reshape(B, N * N, H)
    a3r = a3_dist.reshape(B, N * N, H)

    gath = _sc_gather(t2, sptf, spf, epf, rtab, B, interpret=interpret)
    return _assemble(attn_bias, d2r, a3r, gath, d2_W, a3_W, bias2,
                     t_col, B, H, N, interpret=interpret)
